# baseline (speedup 1.0000x reference)
import functools

import jax
import jax.numpy as jnp
from jax import lax
from jax.experimental import pallas as pl
from jax.experimental.pallas import tpu as pltpu
from jax.experimental.pallas import tpu_sc as plsc

N_NODES = 10000
FDIM = 128
N_GRAPHS = 32

NC = 2
NS = 16
N_TILES = NC * NS

N_PAD = 10112
ROWS_PER_TILE = N_PAD // NS

E_EDGES = 320000
CHUNK = 256
N_CHUNKS = 1280
E_PAD = N_CHUNKS * CHUNK
CHUNKS_PER_TILE = N_CHUNKS // N_TILES

DEG_W = 128

_MESH = plsc.VectorSubcoreMesh(
    core_axis_name="c", subcore_axis_name="s", num_cores=NC, num_subcores=NS
)

_HIGHEST = lax.Precision.HIGHEST


def _dot(a, b):
    return lax.dot_general(a, b, (((1,), (0,)), ((), ())),
                           precision=_HIGHEST, preferred_element_type=jnp.float32)


def _sc_degree(dst_chunks, ones_rows, zeros_deg):
    @functools.partial(
        pl.kernel,
        out_type=jax.ShapeDtypeStruct((NC, N_PAD, DEG_W), jnp.float32),
        mesh=_MESH,
        scratch_types=[
            pltpu.VMEM((CHUNK,), jnp.int32),
            pltpu.VMEM((CHUNK, DEG_W), jnp.float32),
            pltpu.VMEM_SHARED((N_PAD, DEG_W), jnp.float32),
        ],
    )
    def k(dst_hbm, ones_hbm, z_hbm, out_hbm, idx_v, ones_v, acc):
        c = lax.axis_index("c")
        s = lax.axis_index("s")
        wid = s * NC + c
        row0 = s * ROWS_PER_TILE
        pltpu.sync_copy(z_hbm.at[pl.ds(row0, ROWS_PER_TILE)],
                        acc.at[pl.ds(row0, ROWS_PER_TILE)])
        pltpu.sync_copy(ones_hbm, ones_v)
        plsc.subcore_barrier()

        @pl.loop(0, CHUNKS_PER_TILE)
        def _(j):
            chunk = wid * CHUNKS_PER_TILE + j
            pltpu.sync_copy(dst_hbm.at[pl.ds(chunk * CHUNK, CHUNK)], idx_v)
            pltpu.sync_copy(ones_v, acc.at[idx_v], add=True)

        plsc.subcore_barrier()
        pltpu.sync_copy(acc.at[pl.ds(row0, ROWS_PER_TILE)],
                        out_hbm.at[c, pl.ds(row0, ROWS_PER_TILE)])

    return k(dst_chunks, ones_rows, zeros_deg)


def _sc_scatter(g, src_chunks, dst_chunks, zeros_feat):
    @functools.partial(
        pl.kernel,
        out_type=jax.ShapeDtypeStruct((NC, N_PAD, FDIM), jnp.float32),
        mesh=_MESH,
        scratch_types=[
            pltpu.VMEM((CHUNK,), jnp.int32),
            pltpu.VMEM((CHUNK,), jnp.int32),
            pltpu.VMEM((CHUNK, FDIM), jnp.float32),
            pltpu.VMEM_SHARED((N_PAD, FDIM), jnp.float32),
            pltpu.SemaphoreType.DMA,
        ],
    )
    def k(g_hbm, src_hbm, dst_hbm, z_hbm, out_hbm, si_v, di_v, rows_v, acc, sem):
        c = lax.axis_index("c")
        s = lax.axis_index("s")
        wid = s * NC + c
        row0 = s * ROWS_PER_TILE
        pltpu.sync_copy(z_hbm.at[pl.ds(row0, ROWS_PER_TILE)],
                        acc.at[pl.ds(row0, ROWS_PER_TILE)])
        plsc.subcore_barrier()

        @pl.loop(0, CHUNKS_PER_TILE)
        def _(j):
            chunk = wid * CHUNKS_PER_TILE + j
            pltpu.sync_copy(src_hbm.at[pl.ds(chunk * CHUNK, CHUNK)], si_v)
            pltpu.sync_copy(dst_hbm.at[pl.ds(chunk * CHUNK, CHUNK)], di_v)
            pltpu.async_copy(g_hbm.at[si_v], rows_v, sem).wait()
            pltpu.sync_copy(rows_v, acc.at[di_v], add=True)

        plsc.subcore_barrier()
        pltpu.sync_copy(acc.at[pl.ds(row0, ROWS_PER_TILE)],
                        out_hbm.at[c, pl.ds(row0, ROWS_PER_TILE)])

    return k(g, src_chunks, dst_chunks, zeros_feat)


def _tc_matmul(x, W):
    def body(x_ref, w_ref, o_ref):
        o_ref[...] = _dot(x_ref[...], w_ref[...])

    return pl.pallas_call(
        body, out_shape=jax.ShapeDtypeStruct((N_PAD, FDIM), jnp.float32)
    )(x, W)


def _dis_from(degp_ref):
    deg = degp_ref[0, :, 0:1] + degp_ref[1, :, 0:1] + 1.0
    return lax.rsqrt(jnp.maximum(deg, 1.0))


def _tc_first_scale(degp, u):
    def body(degp_ref, u_ref, o_ref):
        o_ref[...] = _dis_from(degp_ref) * u_ref[...]

    return pl.pallas_call(
        body, out_shape=jax.ShapeDtypeStruct((N_PAD, FDIM), jnp.float32)
    )(degp, u)


def _tc_layer(degp, S, g, b, W_next):
    def body(degp_ref, s_ref, g_ref, b_ref, w_ref, o_ref):
        dis = _dis_from(degp_ref)
        h = dis * (s_ref[0] + s_ref[1] + g_ref[...]) + b_ref[...]
        h = jnp.maximum(h, 0.0)
        o_ref[...] = dis * _dot(h, w_ref[...])

    return pl.pallas_call(
        body, out_shape=jax.ShapeDtypeStruct((N_PAD, FDIM), jnp.float32)
    )(degp, S, g, b.reshape(1, FDIM), W_next)


def _tc_head(degp, S, g, b, batch_row, Wl1, bl1, Wl2, bl2):
    def body(degp_ref, s_ref, g_ref, b_ref, batch_ref, wl1_ref, bl1_ref,
             wl2_ref, bl2_ref, o_ref):
        dis = _dis_from(degp_ref)
        h = dis * (s_ref[0] + s_ref[1] + g_ref[...]) + b_ref[...]
        gids = lax.broadcasted_iota(jnp.int32, (N_GRAPHS, N_PAD), 0)
        onehot = jnp.where(batch_ref[...] == gids, 1.0, 0.0)
        sums = _dot(onehot, h)
        counts = jnp.sum(onehot, axis=1, keepdims=True)
        pooled = sums / jnp.maximum(counts, 1.0)
        q = jnp.maximum(_dot(pooled, wl1_ref[...]) + bl1_ref[...], 0.0)
        o_ref[...] = _dot(q, wl2_ref[...]) + bl2_ref[...]

    return pl.pallas_call(
        body, out_shape=jax.ShapeDtypeStruct((N_GRAPHS, 10), jnp.float32)
    )(degp, S, g, b.reshape(1, FDIM), batch_row,
      Wl1, bl1.reshape(1, -1), Wl2, bl2.reshape(1, -1))


def kernel(x, edge_index, batch, W1, b1, W2, b2, W3, b3, W4, b4,
           Wl1, bl1, Wl2, bl2):
    pad_e = jnp.full((E_PAD - E_EDGES,), N_NODES, jnp.int32)
    src_chunks = jnp.concatenate([edge_index[0], pad_e])
    dst_chunks = jnp.concatenate([edge_index[1], pad_e])

    x_pad = jnp.pad(x, ((0, N_PAD - N_NODES), (0, 0)))
    batch_row = jnp.concatenate(
        [batch, jnp.full((N_PAD - N_NODES,), N_GRAPHS, jnp.int32)]
    ).reshape(1, N_PAD)

    zeros_feat = jnp.zeros((N_PAD, FDIM), jnp.float32)
    ones_rows = jnp.ones((CHUNK, DEG_W), jnp.float32)

    degp = _sc_degree(dst_chunks, ones_rows, zeros_feat)
    u1 = _tc_matmul(x_pad, W1)
    g = _tc_first_scale(degp, u1)

    for b, W_next in ((b1, W2), (b2, W3), (b3, W4)):
        S = _sc_scatter(g, src_chunks, dst_chunks, zeros_feat)
        g = _tc_layer(degp, S, g, b, W_next)

    S = _sc_scatter(g, src_chunks, dst_chunks, zeros_feat)
    return _tc_head(degp, S, g, b4, batch_row, Wl1, bl1, Wl2, bl2)


# speedup vs baseline: 1.5923x; 1.5923x over previous
import functools

import jax
import jax.numpy as jnp
from jax import lax
from jax.experimental import pallas as pl
from jax.experimental.pallas import tpu as pltpu
from jax.experimental.pallas import tpu_sc as plsc

N_NODES = 10000
FDIM = 128
N_GRAPHS = 32

NC = 2
NS = 16
N_TILES = NC * NS

N_PAD = 10112
ROWS_PER_TILE = N_PAD // NS

E_EDGES = 320000
CHUNK = 128
N_CHUNKS = 2528
E_PAD = N_CHUNKS * CHUNK
CHUNKS_PER_TILE = N_CHUNKS // N_TILES

PAIR_CHUNKS = 2 * CHUNKS_PER_TILE
CPT0 = 98
CPT1 = PAIR_CHUNKS - CPT0

DEG_W = 128

_MESH = plsc.VectorSubcoreMesh(
    core_axis_name="c", subcore_axis_name="s", num_cores=NC, num_subcores=NS
)

_HIGHEST = lax.Precision.HIGHEST


def _dot(a, b):
    return lax.dot_general(a, b, (((1,), (0,)), ((), ())),
                           precision=_HIGHEST, preferred_element_type=jnp.float32)


def _sc_degree(dst_chunks, ones_rows, zeros_deg):
    @functools.partial(
        pl.kernel,
        out_type=jax.ShapeDtypeStruct((NC, N_PAD, DEG_W), jnp.float32),
        mesh=_MESH,
        scratch_types=[
            pltpu.VMEM((CHUNK,), jnp.int32),
            pltpu.VMEM((CHUNK, DEG_W), jnp.float32),
            pltpu.VMEM_SHARED((N_PAD, DEG_W), jnp.float32),
        ],
    )
    def k(dst_hbm, ones_hbm, z_hbm, out_hbm, idx_v, ones_v, acc):
        c = lax.axis_index("c")
        s = lax.axis_index("s")
        wid = s * NC + c
        row0 = s * ROWS_PER_TILE
        pltpu.sync_copy(z_hbm.at[pl.ds(row0, ROWS_PER_TILE)],
                        acc.at[pl.ds(row0, ROWS_PER_TILE)])
        pltpu.sync_copy(ones_hbm, ones_v)
        plsc.subcore_barrier()

        @pl.loop(0, CHUNKS_PER_TILE)
        def _(j):
            chunk = wid * CHUNKS_PER_TILE + j
            pltpu.sync_copy(dst_hbm.at[pl.ds(chunk * CHUNK, CHUNK)], idx_v)
            pltpu.sync_copy(ones_v, acc.at[idx_v], add=True)

        plsc.subcore_barrier()
        pltpu.sync_copy(acc.at[pl.ds(row0, ROWS_PER_TILE)],
                        out_hbm.at[c, pl.ds(row0, ROWS_PER_TILE)])

    return k(dst_chunks, ones_rows, zeros_deg)


def _sc_scatter(g, src_chunks, dst_chunks, zeros_feat):
    @functools.partial(
        pl.kernel,
        out_type=jax.ShapeDtypeStruct((NC, N_PAD, FDIM), jnp.float32),
        mesh=_MESH,
        scratch_types=[
            pltpu.VMEM((CHUNK,), jnp.int32),
            pltpu.VMEM((CHUNK,), jnp.int32),
            pltpu.VMEM((CHUNK, FDIM), jnp.float32),
            pltpu.VMEM_SHARED((N_PAD, FDIM), jnp.float32),
            pltpu.SemaphoreType.DMA,
        ],
    )
    def k(g_hbm, src_hbm, dst_hbm, z_hbm, out_hbm, si_v, di_v, rows_v, acc, sem):
        c = lax.axis_index("c")
        s = lax.axis_index("s")
        row0 = s * ROWS_PER_TILE
        pltpu.sync_copy(z_hbm.at[pl.ds(row0, ROWS_PER_TILE)],
                        acc.at[pl.ds(row0, ROWS_PER_TILE)])
        plsc.subcore_barrier()
        base = s * PAIR_CHUNKS

        def do_chunk(chunk):
            pltpu.sync_copy(src_hbm.at[pl.ds(chunk * CHUNK, CHUNK)], si_v)
            pltpu.sync_copy(dst_hbm.at[pl.ds(chunk * CHUNK, CHUNK)], di_v)
            pltpu.async_copy(g_hbm.at[si_v], rows_v, sem).wait()
            pltpu.sync_copy(rows_v, acc.at[di_v], add=True)

        @pl.when(c == 0)
        def _():
            @pl.loop(0, CPT0)
            def _(j):
                do_chunk(base + j)

        @pl.when(c == 1)
        def _():
            @pl.loop(0, CPT1)
            def _(j):
                do_chunk(base + CPT0 + j)

        plsc.subcore_barrier()
        pltpu.sync_copy(acc.at[pl.ds(row0, ROWS_PER_TILE)],
                        out_hbm.at[c, pl.ds(row0, ROWS_PER_TILE)])

    return k(g, src_chunks, dst_chunks, zeros_feat)


def _tc_matmul(x, W):
    def body(x_ref, w_ref, o_ref):
        o_ref[...] = _dot(x_ref[...], w_ref[...])

    return pl.pallas_call(
        body, out_shape=jax.ShapeDtypeStruct((N_PAD, FDIM), jnp.float32)
    )(x, W)


def _dis_from(degp_ref):
    deg = degp_ref[0, :, 0:1] + degp_ref[1, :, 0:1] + 1.0
    return lax.rsqrt(jnp.maximum(deg, 1.0))


def _tc_first_scale(degp, u):
    def body(degp_ref, u_ref, o_ref):
        o_ref[...] = _dis_from(degp_ref) * u_ref[...]

    return pl.pallas_call(
        body, out_shape=jax.ShapeDtypeStruct((N_PAD, FDIM), jnp.float32)
    )(degp, u)


def _tc_layer(degp, S, g, b, W_next):
    def body(degp_ref, s_ref, g_ref, b_ref, w_ref, o_ref):
        dis = _dis_from(degp_ref)
        h = dis * (s_ref[0] + s_ref[1] + g_ref[...]) + b_ref[...]
        h = jnp.maximum(h, 0.0)
        o_ref[...] = dis * _dot(h, w_ref[...])

    return pl.pallas_call(
        body, out_shape=jax.ShapeDtypeStruct((N_PAD, FDIM), jnp.float32)
    )(degp, S, g, b.reshape(1, FDIM), W_next)


def _tc_head(degp, S, g, b, batch_row, Wl1, bl1, Wl2, bl2):
    def body(degp_ref, s_ref, g_ref, b_ref, batch_ref, wl1_ref, bl1_ref,
             wl2_ref, bl2_ref, o_ref):
        dis = _dis_from(degp_ref)
        h = dis * (s_ref[0] + s_ref[1] + g_ref[...]) + b_ref[...]
        gids = lax.broadcasted_iota(jnp.int32, (N_GRAPHS, N_PAD), 0)
        onehot = jnp.where(batch_ref[...] == gids, 1.0, 0.0)
        sums = _dot(onehot, h)
        counts = jnp.sum(onehot, axis=1, keepdims=True)
        pooled = sums / jnp.maximum(counts, 1.0)
        q = jnp.maximum(_dot(pooled, wl1_ref[...]) + bl1_ref[...], 0.0)
        o_ref[...] = _dot(q, wl2_ref[...]) + bl2_ref[...]

    return pl.pallas_call(
        body, out_shape=jax.ShapeDtypeStruct((N_GRAPHS, 10), jnp.float32)
    )(degp, S, g, b.reshape(1, FDIM), batch_row,
      Wl1, bl1.reshape(1, -1), Wl2, bl2.reshape(1, -1))


def kernel(x, edge_index, batch, W1, b1, W2, b2, W3, b3, W4, b4,
           Wl1, bl1, Wl2, bl2):
    pad_e = jnp.full((E_PAD - E_EDGES,), N_NODES, jnp.int32)
    src_chunks = jnp.concatenate([edge_index[0], pad_e])
    dst_chunks = jnp.concatenate([edge_index[1], pad_e])

    x_pad = jnp.pad(x, ((0, N_PAD - N_NODES), (0, 0)))
    batch_row = jnp.concatenate(
        [batch, jnp.full((N_PAD - N_NODES,), N_GRAPHS, jnp.int32)]
    ).reshape(1, N_PAD)

    zeros_feat = jnp.zeros((N_PAD, FDIM), jnp.float32)
    ones_rows = jnp.ones((CHUNK, DEG_W), jnp.float32)

    degp = _sc_degree(dst_chunks, ones_rows, zeros_feat)
    u1 = _tc_matmul(x_pad, W1)
    g = _tc_first_scale(degp, u1)

    for b, W_next in ((b1, W2), (b2, W3), (b3, W4)):
        S = _sc_scatter(g, src_chunks, dst_chunks, zeros_feat)
        g = _tc_layer(degp, S, g, b, W_next)

    S = _sc_scatter(g, src_chunks, dst_chunks, zeros_feat)
    return _tc_head(degp, S, g, b4, batch_row, Wl1, bl1, Wl2, bl2)


# speedup vs baseline: 1.6580x; 1.0413x over previous
import functools

import jax
import jax.numpy as jnp
from jax import lax
from jax.experimental import pallas as pl
from jax.experimental.pallas import tpu as pltpu
from jax.experimental.pallas import tpu_sc as plsc

N_NODES = 10000
FDIM = 128
N_GRAPHS = 32

NC = 2
NS = 16
N_TILES = NC * NS

N_PAD = 10112
ROWS_PER_TILE = N_PAD // NS

E_EDGES = 320000
CHUNK = 128
N_CHUNKS = 2528
E_PAD = N_CHUNKS * CHUNK
CHUNKS_PER_TILE = N_CHUNKS // N_TILES

PAIR_CHUNKS = 2 * CHUNKS_PER_TILE
CPT0 = 106
CPT1 = PAIR_CHUNKS - CPT0

DEG_W = 128

_MESH = plsc.VectorSubcoreMesh(
    core_axis_name="c", subcore_axis_name="s", num_cores=NC, num_subcores=NS
)

_HIGHEST = lax.Precision.HIGHEST


def _dot(a, b):
    return lax.dot_general(a, b, (((1,), (0,)), ((), ())),
                           precision=_HIGHEST, preferred_element_type=jnp.float32)


def _sc_degree(dst_chunks, ones_rows, zeros_deg):
    @functools.partial(
        pl.kernel,
        out_type=jax.ShapeDtypeStruct((NC, N_PAD, DEG_W), jnp.float32),
        mesh=_MESH,
        scratch_types=[
            pltpu.VMEM((CHUNK,), jnp.int32),
            pltpu.VMEM((CHUNK, DEG_W), jnp.float32),
            pltpu.VMEM_SHARED((N_PAD, DEG_W), jnp.float32),
        ],
    )
    def k(dst_hbm, ones_hbm, z_hbm, out_hbm, idx_v, ones_v, acc):
        c = lax.axis_index("c")
        s = lax.axis_index("s")
        wid = s * NC + c
        row0 = s * ROWS_PER_TILE
        pltpu.sync_copy(z_hbm.at[pl.ds(row0, ROWS_PER_TILE)],
                        acc.at[pl.ds(row0, ROWS_PER_TILE)])
        pltpu.sync_copy(ones_hbm, ones_v)
        plsc.subcore_barrier()

        @pl.loop(0, CHUNKS_PER_TILE)
        def _(j):
            chunk = wid * CHUNKS_PER_TILE + j
            pltpu.sync_copy(dst_hbm.at[pl.ds(chunk * CHUNK, CHUNK)], idx_v)
            pltpu.sync_copy(ones_v, acc.at[idx_v], add=True)

        plsc.subcore_barrier()
        pltpu.sync_copy(acc.at[pl.ds(row0, ROWS_PER_TILE)],
                        out_hbm.at[c, pl.ds(row0, ROWS_PER_TILE)])

    return k(dst_chunks, ones_rows, zeros_deg)


def _sc_scatter(g, src_chunks, dst_chunks, zeros_feat):
    @functools.partial(
        pl.kernel,
        out_type=jax.ShapeDtypeStruct((NC, N_PAD, FDIM), jnp.float32),
        mesh=_MESH,
        scratch_types=[
            pltpu.VMEM((CHUNK,), jnp.int32),
            pltpu.VMEM((CHUNK,), jnp.int32),
            pltpu.VMEM((CHUNK, FDIM), jnp.float32),
            pltpu.VMEM_SHARED((N_PAD, FDIM), jnp.float32),
            pltpu.SemaphoreType.DMA,
        ],
    )
    def k(g_hbm, src_hbm, dst_hbm, z_hbm, out_hbm, si_v, di_v, rows_v, acc, sem):
        c = lax.axis_index("c")
        s = lax.axis_index("s")
        row0 = s * ROWS_PER_TILE
        pltpu.sync_copy(z_hbm.at[pl.ds(row0, ROWS_PER_TILE)],
                        acc.at[pl.ds(row0, ROWS_PER_TILE)])
        plsc.subcore_barrier()
        base = s * PAIR_CHUNKS

        def do_chunk(chunk):
            pltpu.sync_copy(src_hbm.at[pl.ds(chunk * CHUNK, CHUNK)], si_v)
            pltpu.sync_copy(dst_hbm.at[pl.ds(chunk * CHUNK, CHUNK)], di_v)
            pltpu.async_copy(g_hbm.at[si_v], rows_v, sem).wait()
            pltpu.sync_copy(rows_v, acc.at[di_v], add=True)

        @pl.when(c == 0)
        def _():
            @pl.loop(0, CPT0)
            def _(j):
                do_chunk(base + j)

        @pl.when(c == 1)
        def _():
            @pl.loop(0, CPT1)
            def _(j):
                do_chunk(base + CPT0 + j)

        plsc.subcore_barrier()
        pltpu.sync_copy(acc.at[pl.ds(row0, ROWS_PER_TILE)],
                        out_hbm.at[c, pl.ds(row0, ROWS_PER_TILE)])

    return k(g, src_chunks, dst_chunks, zeros_feat)


def _tc_matmul(x, W):
    def body(x_ref, w_ref, o_ref):
        o_ref[...] = _dot(x_ref[...], w_ref[...])

    return pl.pallas_call(
        body, out_shape=jax.ShapeDtypeStruct((N_PAD, FDIM), jnp.float32)
    )(x, W)


def _dis_from(degp_ref):
    deg = degp_ref[0, :, 0:1] + degp_ref[1, :, 0:1] + 1.0
    return lax.rsqrt(jnp.maximum(deg, 1.0))


def _tc_first_scale(degp, u):
    def body(degp_ref, u_ref, o_ref):
        o_ref[...] = _dis_from(degp_ref) * u_ref[...]

    return pl.pallas_call(
        body, out_shape=jax.ShapeDtypeStruct((N_PAD, FDIM), jnp.float32)
    )(degp, u)


def _tc_layer(degp, S, g, b, W_next):
    def body(degp_ref, s_ref, g_ref, b_ref, w_ref, o_ref):
        dis = _dis_from(degp_ref)
        h = dis * (s_ref[0] + s_ref[1] + g_ref[...]) + b_ref[...]
        h = jnp.maximum(h, 0.0)
        o_ref[...] = dis * _dot(h, w_ref[...])

    return pl.pallas_call(
        body, out_shape=jax.ShapeDtypeStruct((N_PAD, FDIM), jnp.float32)
    )(degp, S, g, b.reshape(1, FDIM), W_next)


def _tc_head(degp, S, g, b, batch_row, Wl1, bl1, Wl2, bl2):
    def body(degp_ref, s_ref, g_ref, b_ref, batch_ref, wl1_ref, bl1_ref,
             wl2_ref, bl2_ref, o_ref):
        dis = _dis_from(degp_ref)
        h = dis * (s_ref[0] + s_ref[1] + g_ref[...]) + b_ref[...]
        gids = lax.broadcasted_iota(jnp.int32, (N_GRAPHS, N_PAD), 0)
        onehot = jnp.where(batch_ref[...] == gids, 1.0, 0.0)
        sums = _dot(onehot, h)
        counts = jnp.sum(onehot, axis=1, keepdims=True)
        pooled = sums / jnp.maximum(counts, 1.0)
        q = jnp.maximum(_dot(pooled, wl1_ref[...]) + bl1_ref[...], 0.0)
        o_ref[...] = _dot(q, wl2_ref[...]) + bl2_ref[...]

    return pl.pallas_call(
        body, out_shape=jax.ShapeDtypeStruct((N_GRAPHS, 10), jnp.float32)
    )(degp, S, g, b.reshape(1, FDIM), batch_row,
      Wl1, bl1.reshape(1, -1), Wl2, bl2.reshape(1, -1))


def kernel(x, edge_index, batch, W1, b1, W2, b2, W3, b3, W4, b4,
           Wl1, bl1, Wl2, bl2):
    pad_e = jnp.full((E_PAD - E_EDGES,), N_NODES, jnp.int32)
    src_chunks = jnp.concatenate([edge_index[0], pad_e])
    dst_chunks = jnp.concatenate([edge_index[1], pad_e])

    x_pad = jnp.pad(x, ((0, N_PAD - N_NODES), (0, 0)))
    batch_row = jnp.concatenate(
        [batch, jnp.full((N_PAD - N_NODES,), N_GRAPHS, jnp.int32)]
    ).reshape(1, N_PAD)

    zeros_feat = jnp.zeros((N_PAD, FDIM), jnp.float32)
    ones_rows = jnp.ones((CHUNK, DEG_W), jnp.float32)

    degp = _sc_degree(dst_chunks, ones_rows, zeros_feat)
    u1 = _tc_matmul(x_pad, W1)
    g = _tc_first_scale(degp, u1)

    for b, W_next in ((b1, W2), (b2, W3), (b3, W4)):
        S = _sc_scatter(g, src_chunks, dst_chunks, zeros_feat)
        g = _tc_layer(degp, S, g, b, W_next)

    S = _sc_scatter(g, src_chunks, dst_chunks, zeros_feat)
    return _tc_head(degp, S, g, b4, batch_row, Wl1, bl1, Wl2, bl2)
